# baseline (speedup 1.0000x reference)
import functools

import jax
import jax.numpy as jnp
from jax import lax
from jax.experimental import pallas as pl
from jax.experimental.pallas import tpu as pltpu
from jax.experimental.pallas import tpu_sc as plsc

_NUM_CODES = 100000
_NUM_CLUSTERS = 1000
_NUM_PARENTS = 50
_SUB = 32
_BATCH, _HIST = 4096, 200
_N = _BATCH * _HIST
_C = 128
_NCHUNKS = _N // _C


@functools.lru_cache(maxsize=None)
def _build():
    info = plsc.get_sparse_core_info()
    nc, ns = info.num_cores, info.num_subcores
    nw = nc * ns
    chunks_w = _NCHUNKS // nw
    nbuf = 4
    kbatch = 8

    mesh = plsc.VectorSubcoreMesh(core_axis_name="c", subcore_axis_name="s")

    @functools.partial(
        pl.kernel,
        out_type=jax.ShapeDtypeStruct((_N, 4 * _SUB), jnp.float32),
        mesh=mesh,
        compiler_params=pltpu.CompilerParams(use_tc_tiling_on_sc=False),
        scratch_types=[
            pltpu.VMEM((chunks_w, _C), jnp.int32),
            pltpu.VMEM((chunks_w, _C), jnp.int32),
            pltpu.VMEM((nbuf, _C, _SUB), jnp.float32),
            pltpu.VMEM((nbuf, _C, 2 * _SUB), jnp.float32),
            pltpu.SemaphoreType.DMA,
            [pltpu.SemaphoreType.DMA] * nbuf,
            [pltpu.SemaphoreType.DMA] * nbuf,
        ],
    )
    def enc(codes2_hbm, m_hbm, cemb_hbm, clp_hbm,
            out_hbm, codes_v, cpid_v, crow_v, cprow_v,
            gsem, rsems, wsems):
        wid = lax.axis_index("s") * nc + lax.axis_index("c")
        g0 = wid * chunks_w

        with jax.named_scope("p1_codes"):
            pltpu.sync_copy(codes2_hbm.at[pl.ds(g0, chunks_w), :], codes_v)

        def id_drain():
            for _ in range(kbatch):
                pltpu.make_async_copy(
                    m_hbm.at[codes_v.at[0]], cpid_v.at[0], gsem).wait()

        def id_batch(t, carry):
            for b in range(kbatch):
                g = t * kbatch + b
                pltpu.async_copy(m_hbm.at[codes_v.at[g]], cpid_v.at[g], gsem)
            @pl.when(t != 0)
            def _():
                id_drain()
            return carry
        with jax.named_scope("p2_ids"):
            lax.fori_loop(0, chunks_w // kbatch, id_batch, 0)
            id_drain()

        def out_band(base, k):
            return out_hbm.at[pl.ds(base, _C), pl.ds(k * _SUB, _SUB)]

        def out_band2(base):
            return out_hbm.at[pl.ds(base, _C), pl.ds(_SUB, 2 * _SUB)]

        def wait_writes(b):
            pltpu.make_async_copy(crow_v.at[b], out_band(0, 0), wsems[b]).wait()
            pltpu.make_async_copy(cprow_v.at[b], out_band2(0), wsems[b]).wait()

        def row_batch(t, carry):
            gds = []
            for b in range(nbuf):
                g = t * nbuf + b
                @pl.when(t != 0)
                def _(b=b):
                    wait_writes(b)
                gds.append(pltpu.async_copy(
                    cemb_hbm.at[codes_v.at[g]], crow_v.at[b], rsems[b]))
                gds.append(pltpu.async_copy(
                    clp_hbm.at[cpid_v.at[g]], cprow_v.at[b], rsems[b]))
            for b in range(nbuf):
                g = t * nbuf + b
                base = (g0 + g) * _C
                gds[2 * b].wait()
                pltpu.async_copy(crow_v.at[b], out_band(base, 0), wsems[b])
                gds[2 * b + 1].wait()
                pltpu.async_copy(cprow_v.at[b], out_band2(base), wsems[b])
            return carry
        with jax.named_scope("p3_rows"):
            lax.fori_loop(0, chunks_w // nbuf, row_batch, 0)
            for b in range(nbuf):
                wait_writes(b)

    return enc


@functools.lru_cache(maxsize=None)
def _build_slice():
    blk = 64

    def body(i_ref, o_ref):
        o_ref[...] = i_ref[:, :, :3 * _SUB]

    return pl.pallas_call(
        body,
        grid=(_BATCH // blk,),
        in_specs=[pl.BlockSpec((blk, _HIST, 4 * _SUB), lambda i: (i, 0, 0))],
        out_specs=pl.BlockSpec((blk, _HIST, 3 * _SUB), lambda i: (i, 0, 0)),
        out_shape=jax.ShapeDtypeStruct((_BATCH, _HIST, 3 * _SUB),
                                       jnp.float32),
    )


def kernel(codes, code_to_cluster, code_to_parent, code_emb, cluster_emb,
           parent_emb):
    codes2 = codes.reshape(_NCHUNKS, _C)
    m = code_to_cluster * _NUM_PARENTS + code_to_parent
    clp = jnp.concatenate([
        jnp.broadcast_to(cluster_emb[:, None, :],
                         (_NUM_CLUSTERS, _NUM_PARENTS, _SUB)),
        jnp.broadcast_to(parent_emb[None, :, :],
                         (_NUM_CLUSTERS, _NUM_PARENTS, _SUB)),
    ], axis=-1).reshape(_NUM_CLUSTERS * _NUM_PARENTS, 2 * _SUB)
    out = _build()(codes2, m, code_emb, clp)
    return _build_slice()(out.reshape(_BATCH, _HIST, 4 * _SUB))


# speedup vs baseline: 1.5452x; 1.5452x over previous
import functools

import jax
import jax.numpy as jnp
from jax import lax
from jax.experimental import pallas as pl
from jax.experimental.pallas import tpu as pltpu
from jax.experimental.pallas import tpu_sc as plsc

_NUM_CODES = 100000
_NUM_CLUSTERS = 1000
_NUM_PARENTS = 50
_SUB = 32
_BATCH, _HIST = 4096, 200
_N = _BATCH * _HIST
_C = 128
_NCHUNKS = _N // _C


@functools.lru_cache(maxsize=None)
def _build():
    info = plsc.get_sparse_core_info()
    nc, ns = info.num_cores, info.num_subcores
    nw = nc * ns
    chunks_w = _NCHUNKS // nw
    nbuf = 5
    kbatch = 8

    mesh = plsc.VectorSubcoreMesh(core_axis_name="c", subcore_axis_name="s")

    @functools.partial(
        pl.kernel,
        out_type=jax.ShapeDtypeStruct((_N, 4 * _SUB), jnp.float32),
        mesh=mesh,
        compiler_params=pltpu.CompilerParams(use_tc_tiling_on_sc=False),
        scratch_types=[
            pltpu.VMEM((chunks_w, _C), jnp.int32),
            pltpu.VMEM((chunks_w, _C), jnp.int32),
            pltpu.VMEM((nbuf, _C, _SUB), jnp.float32),
            pltpu.VMEM((nbuf, _C, 2 * _SUB), jnp.float32),
            pltpu.SemaphoreType.DMA,
            [pltpu.SemaphoreType.DMA] * nbuf,
            [pltpu.SemaphoreType.DMA] * nbuf,
        ],
    )
    def enc(codes2_hbm, m_hbm, cemb_hbm, clp_hbm,
            out_hbm, codes_v, cpid_v, crow_v, cprow_v,
            gsem, rsems, wsems):
        wid = lax.axis_index("s") * nc + lax.axis_index("c")
        g0 = wid * chunks_w

        with jax.named_scope("p1_codes"):
            pltpu.sync_copy(codes2_hbm.at[pl.ds(g0, chunks_w), :], codes_v)

        def id_drain():
            for _ in range(kbatch):
                pltpu.make_async_copy(
                    m_hbm.at[codes_v.at[0]], cpid_v.at[0], gsem).wait()

        def id_batch(t, carry):
            for b in range(kbatch):
                g = t * kbatch + b
                pltpu.async_copy(m_hbm.at[codes_v.at[g]], cpid_v.at[g], gsem)
            @pl.when(t != 0)
            def _():
                id_drain()
            return carry
        with jax.named_scope("p2_ids"):
            lax.fori_loop(0, chunks_w // kbatch, id_batch, 0)
            id_drain()

        def out_band(base, k):
            return out_hbm.at[pl.ds(base, _C), pl.ds(k * _SUB, _SUB)]

        def out_band2(base):
            return out_hbm.at[pl.ds(base, _C), pl.ds(_SUB, 2 * _SUB)]

        def wait_writes(b):
            pltpu.make_async_copy(crow_v.at[b], out_band(0, 0), wsems[b]).wait()
            pltpu.make_async_copy(cprow_v.at[b], out_band2(0), wsems[b]).wait()

        def row_batch(t, carry):
            gds = []
            for b in range(nbuf):
                g = t * nbuf + b
                @pl.when(t != 0)
                def _(b=b):
                    wait_writes(b)
                gds.append(pltpu.async_copy(
                    cemb_hbm.at[codes_v.at[g]], crow_v.at[b], rsems[b]))
                gds.append(pltpu.async_copy(
                    clp_hbm.at[cpid_v.at[g]], cprow_v.at[b], rsems[b]))
            for b in range(nbuf):
                g = t * nbuf + b
                base = (g0 + g) * _C
                gds[2 * b].wait()
                pltpu.async_copy(crow_v.at[b], out_band(base, 0), wsems[b])
                gds[2 * b + 1].wait()
                pltpu.async_copy(cprow_v.at[b], out_band2(base), wsems[b])
            return carry
        with jax.named_scope("p3_rows"):
            lax.fori_loop(0, chunks_w // nbuf, row_batch, 0)
            for b in range(nbuf):
                wait_writes(b)

    return enc


def kernel(codes, code_to_cluster, code_to_parent, code_emb, cluster_emb,
           parent_emb):
    codes2 = codes.reshape(_NCHUNKS, _C)
    m = code_to_cluster * _NUM_PARENTS + code_to_parent
    clp = jnp.concatenate([
        jnp.broadcast_to(cluster_emb[:, None, :],
                         (_NUM_CLUSTERS, _NUM_PARENTS, _SUB)),
        jnp.broadcast_to(parent_emb[None, :, :],
                         (_NUM_CLUSTERS, _NUM_PARENTS, _SUB)),
    ], axis=-1).reshape(_NUM_CLUSTERS * _NUM_PARENTS, 2 * _SUB)
    out = _build()(codes2, m, code_emb, clp)
    return out.reshape(_BATCH, _HIST, 4 * _SUB)[:, :, :3 * _SUB]


# speedup vs baseline: 1.5476x; 1.0015x over previous
import functools

import jax
import jax.numpy as jnp
from jax import lax
from jax.experimental import pallas as pl
from jax.experimental.pallas import tpu as pltpu
from jax.experimental.pallas import tpu_sc as plsc

_NUM_CODES = 100000
_NUM_CLUSTERS = 1000
_NUM_PARENTS = 50
_SUB = 32
_BATCH, _HIST = 4096, 200
_N = _BATCH * _HIST
_C = 128
_NCHUNKS = _N // _C


@functools.lru_cache(maxsize=None)
def _build():
    info = plsc.get_sparse_core_info()
    nc, ns = info.num_cores, info.num_subcores
    nw = nc * ns
    chunks_w = _NCHUNKS // nw
    nbuf = 4
    kbatch = 8

    mesh = plsc.VectorSubcoreMesh(core_axis_name="c", subcore_axis_name="s")

    @functools.partial(
        pl.kernel,
        out_type=jax.ShapeDtypeStruct((_N, 4 * _SUB), jnp.float32),
        mesh=mesh,
        compiler_params=pltpu.CompilerParams(use_tc_tiling_on_sc=False),
        scratch_types=[
            pltpu.VMEM((chunks_w, _C), jnp.int32),
            pltpu.VMEM((chunks_w, _C), jnp.int32),
            pltpu.VMEM((nbuf, _C, _SUB), jnp.float32),
            pltpu.VMEM((nbuf, _C, 2 * _SUB), jnp.float32),
            pltpu.SemaphoreType.DMA,
            [pltpu.SemaphoreType.DMA] * nbuf,
            [pltpu.SemaphoreType.DMA] * nbuf,
        ],
    )
    def enc(codes2_hbm, m_hbm, cemb_hbm, clp_hbm,
            out_hbm, codes_v, cpid_v, crow_v, cprow_v,
            gsem, rsems, wsems):
        wid = lax.axis_index("s") * nc + lax.axis_index("c")
        g0 = wid * chunks_w

        with jax.named_scope("p1_codes"):
            pltpu.sync_copy(codes2_hbm.at[pl.ds(g0, chunks_w), :], codes_v)

        def id_drain():
            for _ in range(kbatch):
                pltpu.make_async_copy(
                    m_hbm.at[codes_v.at[0]], cpid_v.at[0], gsem).wait()

        def id_batch(t, carry):
            for b in range(kbatch):
                g = t * kbatch + b
                pltpu.async_copy(m_hbm.at[codes_v.at[g]], cpid_v.at[g], gsem)
            @pl.when(t != 0)
            def _():
                id_drain()
            return carry
        with jax.named_scope("p2_ids"):
            lax.fori_loop(0, chunks_w // kbatch, id_batch, 0)
            id_drain()

        def out_band(base, k):
            return out_hbm.at[pl.ds(base, _C), pl.ds(k * _SUB, _SUB)]

        def out_band2(base):
            return out_hbm.at[pl.ds(base, _C), pl.ds(_SUB, 2 * _SUB)]

        def wait_writes(b):
            pltpu.make_async_copy(crow_v.at[b], out_band(0, 0), wsems[b]).wait()
            pltpu.make_async_copy(cprow_v.at[b], out_band2(0), wsems[b]).wait()

        def row_batch(t, carry):
            gds = []
            for b in range(nbuf):
                g = t * nbuf + b
                @pl.when(t != 0)
                def _(b=b):
                    wait_writes(b)
                gds.append(pltpu.async_copy(
                    cemb_hbm.at[codes_v.at[g]], crow_v.at[b], rsems[b]))
                gds.append(pltpu.async_copy(
                    clp_hbm.at[cpid_v.at[g]], cprow_v.at[b], rsems[b]))
            for b in range(nbuf):
                g = t * nbuf + b
                base = (g0 + g) * _C
                gds[2 * b].wait()
                pltpu.async_copy(crow_v.at[b], out_band(base, 0), wsems[b])
                gds[2 * b + 1].wait()
                pltpu.async_copy(cprow_v.at[b], out_band2(base), wsems[b])
            return carry
        with jax.named_scope("p3_rows"):
            lax.fori_loop(0, chunks_w // nbuf, row_batch, 0)
            for b in range(nbuf):
                wait_writes(b)

    return enc


def kernel(codes, code_to_cluster, code_to_parent, code_emb, cluster_emb,
           parent_emb):
    codes2 = codes.reshape(_NCHUNKS, _C)
    m = code_to_cluster * _NUM_PARENTS + code_to_parent
    clp = jnp.concatenate([
        jnp.broadcast_to(cluster_emb[:, None, :],
                         (_NUM_CLUSTERS, _NUM_PARENTS, _SUB)),
        jnp.broadcast_to(parent_emb[None, :, :],
                         (_NUM_CLUSTERS, _NUM_PARENTS, _SUB)),
    ], axis=-1).reshape(_NUM_CLUSTERS * _NUM_PARENTS, 2 * _SUB)
    out = _build()(codes2, m, code_emb, clp)
    return out.reshape(_BATCH, _HIST, 4 * _SUB)[:, :, :3 * _SUB]
